# baseline (speedup 1.0000x reference)
import functools

import jax
import jax.numpy as jnp
from jax import lax
from jax.experimental import pallas as pl
from jax.experimental.pallas import tpu as pltpu
from jax.experimental.pallas import tpu_sc as plsc

B, F, D = 128, 16, 1000
L = 16
NC, NS = 2, 16
NW = NC * NS
TP = 32

_mesh = plsc.VectorSubcoreMesh(core_axis_name="c", subcore_axis_name="s")


HALF = TP // 2


@functools.partial(
    pl.kernel,
    mesh=_mesh,
    out_type=jax.ShapeDtypeStruct((D, B), jnp.float32),
    scratch_types=[
        pltpu.VMEM((2, F, HALF, B), jnp.float32),
        pltpu.VMEM((2, HALF, B), jnp.float32),
        pltpu.SemaphoreType.DMA,
        pltpu.SemaphoreType.DMA,
        pltpu.SemaphoreType.DMA,
    ],
)
def _mean_sc(x_hbm, out_hbm, x_v, o_v, s0, s1, so):
    wid = lax.axis_index("s") * NC + lax.axis_index("c")
    tile = jnp.minimum(wid * (TP // 8), D // 8 - TP // 8)
    start = pl.multiple_of(tile * 8, 8)

    def fetch(h, sem):
        hs = pl.multiple_of(start + h * HALF, 8)
        return [
            pltpu.async_copy(x_hbm.at[f, pl.ds(hs, HALF), :], x_v.at[h, f], sem)
            for f in range(F)
        ]

    def reduce_half(h):
        def row(r, carry):
            for c in range(B // L):
                sl = pl.ds(c * L, L)
                acc0 = x_v[h, 0, r, sl] + x_v[h, 1, r, sl]
                acc1 = x_v[h, 2, r, sl] + x_v[h, 3, r, sl]
                acc2 = x_v[h, 4, r, sl] + x_v[h, 5, r, sl]
                acc3 = x_v[h, 6, r, sl] + x_v[h, 7, r, sl]
                for f in range(8, F):
                    acc0 = acc0 + x_v[h, f, r, sl]
                o_v[h, r, sl] = ((acc0 + acc1) + (acc2 + acc3)) * (1.0 / F)
            return carry

        lax.fori_loop(0, HALF, row, 0)

    cp0 = fetch(0, s0)
    cp1 = fetch(1, s1)
    out_cps = []
    for h, cps in ((0, cp0), (1, cp1)):
        for cp in cps:
            cp.wait()
        reduce_half(h)
        hs = pl.multiple_of(start + h * HALF, 8)
        out_cps.append(pltpu.async_copy(o_v.at[h], out_hbm.at[pl.ds(hs, HALF), :], so))
    for cp in out_cps:
        cp.wait()


def kernel(input):
    x_t = jnp.transpose(input, (1, 2, 0))
    out_t = _mean_sc(x_t)
    return jnp.transpose(out_t)[:, None, :]
